# speedup vs baseline: 2.4355x; 2.4355x over previous
import functools
import math

import jax
import jax.numpy as jnp
from jax import lax
from jax.experimental import pallas as pl
from jax.experimental.pallas import tpu as pltpu
from jax.experimental.pallas import tpu_sc as plsc

SMOOTH = 0.1
CONF = 1.0 - SMOOTH

_NC = 2
_NS = 16
_NW = _NC * _NS
_LANES = 16


def _sc_gather_body(v, per_w, flat_hbm, tgt_hbm, out_hbm,
                    tgt_v, idx_v, vals_v, sem):
    wid = lax.axis_index("s") * _NC + lax.axis_index("c")
    base = wid * per_w
    pltpu.sync_copy(tgt_hbm.at[pl.ds(base, per_w)], tgt_v)
    for c in range(per_w // _LANES):
        t16 = tgt_v[pl.ds(c * _LANES, _LANES)]
        rows = lax.broadcasted_iota(jnp.int32, (_LANES,), 0)
        row0 = base + c * _LANES
        idx_v[pl.ds(c * _LANES, _LANES)] = (rows + row0) * v + t16
    pltpu.async_copy(flat_hbm.at[idx_v], vals_v, sem).wait()
    pltpu.sync_copy(vals_v, out_hbm.at[pl.ds(base, per_w)])


def _make_sc_gather(n_rows, v):
    per_w = n_rows // _NW
    mesh = plsc.VectorSubcoreMesh(core_axis_name="c", subcore_axis_name="s")
    return pl.kernel(
        functools.partial(_sc_gather_body, v, per_w),
        out_type=jax.ShapeDtypeStruct((n_rows,), jnp.float32),
        mesh=mesh,
        scratch_types=[
            pltpu.VMEM((per_w,), jnp.int32),
            pltpu.VMEM((per_w,), jnp.int32),
            pltpu.VMEM((per_w,), jnp.float32),
            pltpu.SemaphoreType.DMA,
        ],
    )


def _tc_body(nsteps_i, nsteps_j, c_const, eps,
             x_ref, m_ref, g_ref, out_ref, acc_s, acc_g, acc_m):
    i = pl.program_id(0)
    j = pl.program_id(1)

    @pl.when((i == 0) & (j == 0))
    def _init():
        acc_s[0, 0] = 0.0
        acc_g[0, 0] = 0.0
        acc_m[0, 0] = 0.0

    x = x_ref[...]
    m = m_ref[...]
    rowsum = jnp.sum(x, axis=1, keepdims=True)
    acc_s[0, 0] += jnp.sum(rowsum * m)

    @pl.when(j == 0)
    def _per_rowblock():
        acc_g[0, 0] += jnp.sum(m * g_ref[...])
        acc_m[0, 0] += jnp.sum(m)

    @pl.when((i == nsteps_i - 1) & (j == nsteps_j - 1))
    def _fin():
        mt = acc_m[0, 0]
        out_ref[0, 0] = (c_const * mt - eps * acc_s[0, 0]
                         + (eps - CONF) * acc_g[0, 0]) / mt


def _make_tc_loss(n_rows, v, block_r, block_w):
    ni = n_rows // block_r
    nj = v // block_w
    eps = SMOOTH / (v - 1)
    c_const = (v - 1) * eps * math.log(eps) + CONF * math.log(CONF)
    return pl.pallas_call(
        functools.partial(_tc_body, ni, nj, c_const, eps),
        grid=(ni, nj),
        in_specs=[
            pl.BlockSpec((block_r, block_w), lambda i, j: (i, j)),
            pl.BlockSpec((block_r, 1), lambda i, j: (i, 0)),
            pl.BlockSpec((block_r, 1), lambda i, j: (i, 0)),
        ],
        out_specs=pl.BlockSpec((1, 1), lambda i, j: (0, 0),
                               memory_space=pltpu.SMEM),
        out_shape=jax.ShapeDtypeStruct((1, 1), jnp.float32),
        scratch_shapes=[
            pltpu.SMEM((1, 1), jnp.float32),
            pltpu.SMEM((1, 1), jnp.float32),
            pltpu.SMEM((1, 1), jnp.float32),
        ],
    )


def kernel(input, target, mask):
    b, t, v = input.shape
    n = b * t
    x = input.reshape(n, v)
    tgt = target.reshape(n).astype(jnp.int32)
    m = mask.reshape(n, 1)

    g = _make_sc_gather(n, v)(x.reshape(-1), tgt)
    loss = _make_tc_loss(n, v, 256, 6400)(x, m, g.reshape(n, 1))
    return loss[0, 0]
